# Initial kernel scaffold; baseline (speedup 1.0000x reference)
import jax
import jax.numpy as jnp
from jax.experimental import pallas as pl


def kernel(x, W1, b1, W2, b2, w_gate):
    raise NotImplementedError("write your pallas kernel here")


# speedup vs baseline: 2.8075x; 2.8075x over previous
import functools

import jax
import jax.numpy as jnp
from jax.experimental import pallas as pl
from jax.experimental.pallas import tpu as pltpu

INPUT_DIM = 768
OUTPUT_DIM = 768
HIDDEN = 128
NUM_EXPERTS = 64
TOP_K = 2
N_TOKENS = 4096


def _moe_body(x_ref, W1_ref, b1_ref, W2_ref, b2_ref, wg_ref,
              out_ref, aux_ref, g_ref):
    e = pl.program_id(0)

    @pl.when(e == 0)
    def _gating():
        x = x_ref[...]
        logits = jnp.dot(x, wg_ref[...], preferred_element_type=jnp.float32)
        m = jnp.max(logits, axis=1, keepdims=True)
        ex = jnp.exp(logits - m)
        gates = ex / jnp.sum(ex, axis=1, keepdims=True)
        importance = jnp.mean(gates, axis=0)
        tgt = 1.0 / NUM_EXPERTS
        aux = jnp.sum(tgt * (jnp.log(tgt) - jnp.log(importance)))
        aux_ref[...] = aux.reshape(1, 1)
        eids = jax.lax.broadcasted_iota(jnp.int32, logits.shape, 1)
        m0 = jnp.max(logits, axis=1, keepdims=True)
        is0 = logits == m0
        idx0 = jnp.min(jnp.where(is0, eids, NUM_EXPERTS), axis=1, keepdims=True)
        neg = jnp.float32(-jnp.inf)
        logits1 = jnp.where(eids == idx0, neg, logits)
        m1 = jnp.max(logits1, axis=1, keepdims=True)
        is1 = logits1 == m1
        idx1 = jnp.min(jnp.where(is1, eids, NUM_EXPERTS), axis=1, keepdims=True)
        w0 = 1.0 / (1.0 + jnp.exp(m1 - m0))
        w1 = 1.0 - w0
        g_ref[...] = jnp.where(eids == idx0, w0,
                               jnp.where(eids == idx1, w1, 0.0))
        out_ref[...] = jnp.zeros_like(out_ref)

    x = x_ref[...]
    h = jnp.dot(x, W1_ref[0], preferred_element_type=jnp.float32)
    h = jnp.maximum(h + b1_ref[0], 0.0)
    y = jnp.dot(h, W2_ref[0], preferred_element_type=jnp.float32)
    y = y + b2_ref[0]
    g = g_ref[...]
    lane = jax.lax.broadcasted_iota(jnp.int32, g.shape, 1)
    g_col = jnp.sum(jnp.where(lane == e, g, 0.0), axis=1, keepdims=True)
    out_ref[...] += g_col * y


@jax.jit
def kernel(x, W1, b1, W2, b2, w_gate):
    out, aux = pl.pallas_call(
        _moe_body,
        grid=(NUM_EXPERTS,),
        in_specs=[
            pl.BlockSpec((N_TOKENS, INPUT_DIM), lambda e: (0, 0)),
            pl.BlockSpec((1, INPUT_DIM, HIDDEN), lambda e: (e, 0, 0)),
            pl.BlockSpec((1, 1, HIDDEN), lambda e: (e, 0, 0)),
            pl.BlockSpec((1, HIDDEN, OUTPUT_DIM), lambda e: (e, 0, 0)),
            pl.BlockSpec((1, 1, OUTPUT_DIM), lambda e: (e, 0, 0)),
            pl.BlockSpec((INPUT_DIM, NUM_EXPERTS), lambda e: (0, 0)),
        ],
        out_specs=[
            pl.BlockSpec((N_TOKENS, OUTPUT_DIM), lambda e: (0, 0)),
            pl.BlockSpec((1, 1), lambda e: (0, 0)),
        ],
        out_shape=[
            jax.ShapeDtypeStruct((N_TOKENS, OUTPUT_DIM), jnp.float32),
            jax.ShapeDtypeStruct((1, 1), jnp.float32),
        ],
        scratch_shapes=[pltpu.VMEM((N_TOKENS, NUM_EXPERTS), jnp.float32)],
    )(x, W1, b1.reshape(NUM_EXPERTS, 1, HIDDEN),
      W2, b2.reshape(NUM_EXPERTS, 1, OUTPUT_DIM), w_gate)
    return out, aux[0, 0]
